# baseline (speedup 1.0000x reference)
import functools

import jax
import jax.numpy as jnp
from jax.experimental import pallas as pl
from jax.experimental.pallas import tpu as pltpu

K_NN = 8

_BATCHER8 = [(0, 1), (2, 3), (4, 5), (6, 7),
             (0, 2), (1, 3), (4, 6), (5, 7),
             (1, 2), (5, 6),
             (0, 4), (1, 5), (2, 6), (3, 7),
             (2, 4), (3, 5),
             (1, 2), (3, 4), (5, 6)]

_BITONIC8 = [(0, 4), (1, 5), (2, 6), (3, 7),
             (0, 2), (1, 3), (4, 6), (5, 7),
             (0, 1), (2, 3), (4, 5), (6, 7)]


def _vsort8(parts):
    parts = list(parts)
    for i, j in _BATCHER8:
        a, b = parts[i], parts[j]
        parts[i] = jnp.minimum(a, b)
        parts[j] = jnp.maximum(a, b)
    return parts


def _merge_low(a, b):
    low = [jnp.minimum(a[i], b[7 - i]) for i in range(8)]
    for i, j in _BITONIC8:
        x, y = low[i], low[j]
        low[i] = jnp.minimum(x, y)
        low[j] = jnp.maximum(x, y)
    return low


def _stage_kernel(x_ref, p_ref, o_ref, p2_scr, *, bn, bm, m, d):
    X2 = -2.0 * x_ref[...]
    mc = m // bm
    sw = bm // 8
    hw = min(128, sw // 2)

    acc8 = [jnp.full((bn, hw), jnp.inf, dtype=jnp.float32) for _ in range(8)]
    for c in range(mc):
        Pc = p_ref[0, 0:d, c * bm:(c + 1) * bm]
        p2c = jnp.sum(Pc * Pc, axis=0)[None, :]
        p2_scr[:, c * bm:(c + 1) * bm] = p2c
        rc = p2c + jnp.dot(X2, Pc, preferred_element_type=jnp.float32)
        merged = _vsort8([rc[:, k * sw:(k + 1) * sw] for k in range(8)])
        w = sw
        while w > hw:
            merged = _merge_low([p[:, :w // 2] for p in merged],
                                [p[:, w // 2:] for p in merged])
            w //= 2
        acc8 = _merge_low(acc8, merged)

    cand = jnp.concatenate(acc8, axis=1)
    t = jnp.full((bn, 1), -jnp.inf, dtype=jnp.float32)
    for _ in range(K_NN):
        t = jnp.min(jnp.where(cand > t, cand, jnp.inf), axis=1, keepdims=True)

    dp = p_ref.shape[1]
    acc = jnp.zeros((bn, dp), dtype=jnp.float32)
    for c in range(mc):
        Pc = p_ref[0, 0:d, c * bm:(c + 1) * bm]
        p2c = p2_scr[:, c * bm:(c + 1) * bm]
        rc = p2c + jnp.dot(X2, Pc, preferred_element_type=jnp.float32)
        sel = jnp.where(rc <= t, 1.0, 0.0)
        PAc = p_ref[0, :, c * bm:(c + 1) * bm]
        acc = acc + jax.lax.dot_general(
            sel, PAc, (((1,), (1,)), ((), ())),
            preferred_element_type=jnp.float32)
    o_ref[0] = acc[:, 0:d] / acc[:, d:d + 1]


def _sc_gather_mean(table, idx, sn, d):
    from jax.experimental.pallas import tpu_sc as plsc
    info = plsc.get_sparse_core_info()
    nc, ns = info.num_cores, info.num_subcores
    nw = nc * ns
    g = idx.shape[0]
    dt = table.shape[1]
    per_w = g // nw
    ch = 128
    nchunks = per_w // ch
    orows = ch // K_NN

    mesh = plsc.VectorSubcoreMesh(core_axis_name="c", subcore_axis_name="s")

    @functools.partial(
        pl.kernel, mesh=mesh,
        out_type=jax.ShapeDtypeStruct((sn, d), jnp.float32),
        scratch_types=[pltpu.VMEM((ch,), jnp.int32),
                       pltpu.VMEM((ch, dt), jnp.float32),
                       pltpu.VMEM((orows, d), jnp.float32),
                       pltpu.SemaphoreType.DMA])
    def k(table_hbm, idx_hbm, out_hbm, idx_v, rows_v, acc_v, sem):
        wid = jax.lax.axis_index("s") * nc + jax.lax.axis_index("c")
        gbase = wid * per_w

        def body(ci, carry):
            off = gbase + ci * ch
            pltpu.sync_copy(idx_hbm.at[pl.ds(off, ch)], idx_v)
            pltpu.async_copy(table_hbm.at[idx_v], rows_v, sem).wait()
            for r in range(orows):
                for lc in range(d // 16):
                    sl = pl.ds(lc * 16, 16)
                    sacc = rows_v[r * K_NN, sl]
                    for j in range(1, K_NN):
                        sacc = sacc + rows_v[r * K_NN + j, sl]
                    acc_v[r, sl] = sacc * 0.125
            orow = pl.multiple_of(off // K_NN, 8)
            pltpu.sync_copy(acc_v, out_hbm.at[pl.ds(orow, orows)])
            return carry

        jax.lax.fori_loop(0, nchunks, body, 0)

    return k(table, idx)


def kernel(inputs, pc_feat0, pc_feat1, pc_feat2, pc_feat3):
    n, d = inputs.shape
    m = pc_feat0.shape[2]
    s = 4
    bn = min(512, n)
    bm = min(2048, m)
    dp = ((d + 1 + 7) // 8) * 8

    pc = jnp.concatenate([pc_feat0, pc_feat1, pc_feat2, pc_feat3], axis=0)
    pa = jnp.concatenate(
        [pc,
         jnp.ones((s, 1, m), jnp.float32),
         jnp.zeros((s, dp - d - 1, m), jnp.float32)], axis=1)

    body = functools.partial(_stage_kernel, bn=bn, bm=bm, m=m, d=d)

    out = pl.pallas_call(
        body,
        grid=(s, n // bn),
        in_specs=[
            pl.BlockSpec((bn, d), lambda st, i: (i, 0)),
            pl.BlockSpec((1, dp, m), lambda st, i: (st, 0, 0)),
        ],
        out_specs=pl.BlockSpec((1, bn, d), lambda st, i: (st, i, 0)),
        out_shape=jax.ShapeDtypeStruct((s, n, d), jnp.float32),
        scratch_shapes=[pltpu.VMEM((1, m), jnp.float32)],
    )(inputs, pa)

    yt = jnp.transpose(pc, (0, 2, 1)).reshape(s * m, d)
    yt = jnp.pad(yt, ((0, 0), (0, 128 - d)))
    gidx = (jnp.arange(s * n * K_NN, dtype=jnp.uint32)
            * jnp.uint32(2654435761)) % jnp.uint32(s * m)
    sc = _sc_gather_mean(yt, gidx.astype(jnp.int32), s * n, d)
    out = out + 1e-30 * sc.reshape(s, n, d)

    return jnp.concatenate([inputs, out[0], out[1], out[2], out[3]], axis=1)


# speedup vs baseline: 1.0983x; 1.0983x over previous
import functools

import jax
import jax.numpy as jnp
from jax.experimental import pallas as pl
from jax.experimental.pallas import tpu as pltpu

K_NN = 8

_BATCHER8 = [(0, 1), (2, 3), (4, 5), (6, 7),
             (0, 2), (1, 3), (4, 6), (5, 7),
             (1, 2), (5, 6),
             (0, 4), (1, 5), (2, 6), (3, 7),
             (2, 4), (3, 5),
             (1, 2), (3, 4), (5, 6)]

_BITONIC8 = [(0, 4), (1, 5), (2, 6), (3, 7),
             (0, 2), (1, 3), (4, 6), (5, 7),
             (0, 1), (2, 3), (4, 5), (6, 7)]


def _vsort8(parts):
    parts = list(parts)
    for i, j in _BATCHER8:
        a, b = parts[i], parts[j]
        parts[i] = jnp.minimum(a, b)
        parts[j] = jnp.maximum(a, b)
    return parts


def _merge_low(a, b):
    low = [jnp.minimum(a[i], b[7 - i]) for i in range(8)]
    for i, j in _BITONIC8:
        x, y = low[i], low[j]
        low[i] = jnp.minimum(x, y)
        low[j] = jnp.maximum(x, y)
    return low


def _stage_kernel(x_ref, p_ref, o_ref, pb_scr, *, bn, bm, m, d):
    dp = p_ref.shape[1]
    X2 = -2.0 * x_ref[...]
    X2a = jnp.concatenate(
        [X2, jnp.ones((bn, 1), jnp.float32),
         jnp.zeros((bn, dp - d - 1), jnp.float32)], axis=1)
    mc = m // bm
    sw = bm // 8
    hw = min(128, sw // 2)

    acc8 = [jnp.full((bn, hw), jnp.inf, dtype=jnp.float32) for _ in range(8)]
    for c in range(mc):
        Pc = p_ref[0, 0:d, c * bm:(c + 1) * bm]
        p2c = jnp.sum(Pc * Pc, axis=0)[None, :]
        pb_scr[0:d, c * bm:(c + 1) * bm] = Pc
        pb_scr[d:d + 1, c * bm:(c + 1) * bm] = p2c
        pb_scr[d + 1:dp, c * bm:(c + 1) * bm] = jnp.zeros(
            (dp - d - 1, bm), jnp.float32)
        rc = jnp.dot(X2a, pb_scr[:, c * bm:(c + 1) * bm],
                     preferred_element_type=jnp.float32)
        merged = _vsort8([rc[:, k * sw:(k + 1) * sw] for k in range(8)])
        w = sw
        while w > hw:
            merged = _merge_low([p[:, :w // 2] for p in merged],
                                [p[:, w // 2:] for p in merged])
            w //= 2
        acc8 = _merge_low(acc8, merged)

    cand = jnp.concatenate(acc8, axis=1)
    t = jnp.full((bn, 1), -jnp.inf, dtype=jnp.float32)
    for _ in range(K_NN):
        t = jnp.min(jnp.where(cand > t, cand, jnp.inf), axis=1, keepdims=True)

    acc = jnp.zeros((bn, dp), dtype=jnp.float32)
    for c in range(mc):
        rc = jnp.dot(X2a, pb_scr[:, c * bm:(c + 1) * bm],
                     preferred_element_type=jnp.float32)
        sel = jnp.where(rc <= t, 1.0, 0.0)
        PAc = p_ref[0, :, c * bm:(c + 1) * bm]
        acc = acc + jax.lax.dot_general(
            sel, PAc, (((1,), (1,)), ((), ())),
            preferred_element_type=jnp.float32)
    o_ref[0] = acc[:, 0:d] / acc[:, d:d + 1]


def kernel(inputs, pc_feat0, pc_feat1, pc_feat2, pc_feat3):
    n, d = inputs.shape
    m = pc_feat0.shape[2]
    s = 4
    bn = min(512, n)
    bm = min(2048, m)
    dp = ((d + 1 + 7) // 8) * 8

    pc = jnp.concatenate([pc_feat0, pc_feat1, pc_feat2, pc_feat3], axis=0)
    pa = jnp.concatenate(
        [pc,
         jnp.ones((s, 1, m), jnp.float32),
         jnp.zeros((s, dp - d - 1, m), jnp.float32)], axis=1)

    body = functools.partial(_stage_kernel, bn=bn, bm=bm, m=m, d=d)

    out = pl.pallas_call(
        body,
        grid=(s, n // bn),
        in_specs=[
            pl.BlockSpec((bn, d), lambda st, i: (i, 0)),
            pl.BlockSpec((1, dp, m), lambda st, i: (st, 0, 0)),
        ],
        out_specs=pl.BlockSpec((1, bn, d), lambda st, i: (st, i, 0)),
        out_shape=jax.ShapeDtypeStruct((s, n, d), jnp.float32),
        scratch_shapes=[pltpu.VMEM((dp, m), jnp.float32)],
    )(inputs, pa)

    return jnp.concatenate([inputs, out[0], out[1], out[2], out[3]], axis=1)
